# baseline (speedup 1.0000x reference)
import jax
import jax.numpy as jnp
from jax.experimental import pallas as pl

_D_MODEL = 768
_NUM_EXPERTS = 8
_TOP_K = 2
_BLOCK_ROWS = 1024


def _router_body(x_ref, wt_ref, probs_ref, w_ref, idx_ref):
    x_blk = x_ref[...]
    wt = wt_ref[...]
    logits = jnp.dot(x_blk, wt, preferred_element_type=jnp.float32)
    lt = logits.T

    m = jnp.max(lt, axis=0, keepdims=True)
    e = jnp.exp(lt - m)
    denom = jnp.sum(e, axis=0, keepdims=True)
    inv = 1.0 / denom

    iota = jax.lax.broadcasted_iota(jnp.int32, lt.shape, 0)
    i1 = jnp.min(jnp.where(lt == m, iota, _NUM_EXPERTS), axis=0,
                 keepdims=True)
    masked = jnp.where(iota == i1, -jnp.inf, lt)
    m2 = jnp.max(masked, axis=0, keepdims=True)
    i2 = jnp.min(jnp.where(masked == m2, iota, _NUM_EXPERTS), axis=0,
                 keepdims=True)

    p1 = inv
    p2 = jnp.exp(m2 - m) * inv
    wsum = p1 + p2 + 1e-9
    probs_ref[...] = (e * inv).T
    w_ref[...] = jnp.concatenate([p1 / wsum, p2 / wsum], axis=0).T
    idx_ref[...] = jnp.concatenate([i1, i2], axis=0).T


def kernel(x, W):
    B, S, D = x.shape
    E = W.shape[0]
    n = B * S
    xf = x.reshape(n, D)
    wt = W.T

    grid = (n // _BLOCK_ROWS,)
    probs_t, weights_t, idx_t = pl.pallas_call(
        _router_body,
        grid=grid,
        in_specs=[
            pl.BlockSpec((_BLOCK_ROWS, D), lambda i: (i, 0)),
            pl.BlockSpec((D, E), lambda i: (0, 0)),
        ],
        out_specs=[
            pl.BlockSpec((_BLOCK_ROWS, E), lambda i: (i, 0)),
            pl.BlockSpec((_BLOCK_ROWS, _TOP_K), lambda i: (i, 0)),
            pl.BlockSpec((_BLOCK_ROWS, _TOP_K), lambda i: (i, 0)),
        ],
        out_shape=[
            jax.ShapeDtypeStruct((n, E), jnp.float32),
            jax.ShapeDtypeStruct((n, _TOP_K), jnp.float32),
            jax.ShapeDtypeStruct((n, _TOP_K), jnp.int32),
        ],
    )(xf, wt)

    return (weights_t.reshape(B, S, _TOP_K),
            idx_t.reshape(B, S, _TOP_K),
            probs_t.reshape(B, S, E))


# speedup vs baseline: 2.5009x; 2.5009x over previous
import jax
import jax.numpy as jnp
from jax.experimental import pallas as pl

_D_MODEL = 768
_NUM_EXPERTS = 8
_TOP_K = 2
_BLOCK_ROWS = 2048


def _router_body(x_ref, wt_ref, probs_ref, w_ref, idx_ref):
    x_blk = x_ref[...]
    wt = wt_ref[...]
    logits = jnp.dot(x_blk, wt, preferred_element_type=jnp.float32)
    lt = logits.T

    m = jnp.max(lt, axis=0, keepdims=True)
    e = jnp.exp(lt - m)
    denom = jnp.sum(e, axis=0, keepdims=True)
    inv = 1.0 / denom

    iota = jax.lax.broadcasted_iota(jnp.int32, lt.shape, 0)
    i1 = jnp.min(jnp.where(lt == m, iota, _NUM_EXPERTS), axis=0,
                 keepdims=True)
    masked = jnp.where(iota == i1, -jnp.inf, lt)
    m2 = jnp.max(masked, axis=0, keepdims=True)
    i2 = jnp.min(jnp.where(masked == m2, iota, _NUM_EXPERTS), axis=0,
                 keepdims=True)

    p1 = inv
    p2 = jnp.exp(m2 - m) * inv
    wsum = p1 + p2 + 1e-9
    probs_ref[...] = e * inv
    w_ref[...] = jnp.concatenate([p1 / wsum, p2 / wsum], axis=0)
    idx_ref[...] = jnp.concatenate([i1, i2], axis=0)


def kernel(x, W):
    B, S, D = x.shape
    E = W.shape[0]
    n = B * S
    xf = x.reshape(n, D)
    wt = W.T

    grid = (n // _BLOCK_ROWS,)
    probs_t, weights_t, idx_t = pl.pallas_call(
        _router_body,
        grid=grid,
        in_specs=[
            pl.BlockSpec((_BLOCK_ROWS, D), lambda i: (i, 0)),
            pl.BlockSpec((D, E), lambda i: (0, 0)),
        ],
        out_specs=[
            pl.BlockSpec((E, _BLOCK_ROWS), lambda i: (0, i)),
            pl.BlockSpec((_TOP_K, _BLOCK_ROWS), lambda i: (0, i)),
            pl.BlockSpec((_TOP_K, _BLOCK_ROWS), lambda i: (0, i)),
        ],
        out_shape=[
            jax.ShapeDtypeStruct((E, n), jnp.float32),
            jax.ShapeDtypeStruct((_TOP_K, n), jnp.float32),
            jax.ShapeDtypeStruct((_TOP_K, n), jnp.int32),
        ],
    )(xf, wt)

    return (weights_t.T.reshape(B, S, _TOP_K),
            idx_t.T.reshape(B, S, _TOP_K),
            probs_t.T.reshape(B, S, E))


# speedup vs baseline: 2.5788x; 1.0311x over previous
import jax
import jax.numpy as jnp
from jax.experimental import pallas as pl

_D_MODEL = 768
_NUM_EXPERTS = 8
_TOP_K = 2
_BLOCK_ROWS = 4096


def _router_body(x_ref, wt_ref, probs_ref, w_ref, idx_ref):
    x_blk = x_ref[...]
    wt = wt_ref[...]
    logits = jnp.dot(x_blk, wt, preferred_element_type=jnp.float32)
    lt = logits.T

    m = jnp.max(lt, axis=0, keepdims=True)
    e = jnp.exp(lt - m)
    denom = jnp.sum(e, axis=0, keepdims=True)
    inv = 1.0 / denom

    iota = jax.lax.broadcasted_iota(jnp.int32, lt.shape, 0)
    i1 = jnp.min(jnp.where(lt == m, iota, _NUM_EXPERTS), axis=0,
                 keepdims=True)
    masked = jnp.where(iota == i1, -jnp.inf, lt)
    m2 = jnp.max(masked, axis=0, keepdims=True)
    i2 = jnp.min(jnp.where(masked == m2, iota, _NUM_EXPERTS), axis=0,
                 keepdims=True)

    p1 = inv
    p2 = jnp.exp(m2 - m) * inv
    wsum = p1 + p2 + 1e-9
    probs_ref[...] = e * inv
    w_ref[...] = jnp.concatenate([p1 / wsum, p2 / wsum], axis=0)
    idx_ref[...] = jnp.concatenate([i1, i2], axis=0)


def kernel(x, W):
    B, S, D = x.shape
    E = W.shape[0]
    n = B * S
    xf = x.reshape(n, D)
    wt = W.T

    grid = (n // _BLOCK_ROWS,)
    probs_t, weights_t, idx_t = pl.pallas_call(
        _router_body,
        grid=grid,
        in_specs=[
            pl.BlockSpec((_BLOCK_ROWS, D), lambda i: (i, 0)),
            pl.BlockSpec((D, E), lambda i: (0, 0)),
        ],
        out_specs=[
            pl.BlockSpec((E, _BLOCK_ROWS), lambda i: (0, i)),
            pl.BlockSpec((_TOP_K, _BLOCK_ROWS), lambda i: (0, i)),
            pl.BlockSpec((_TOP_K, _BLOCK_ROWS), lambda i: (0, i)),
        ],
        out_shape=[
            jax.ShapeDtypeStruct((E, n), jnp.float32),
            jax.ShapeDtypeStruct((_TOP_K, n), jnp.float32),
            jax.ShapeDtypeStruct((_TOP_K, n), jnp.int32),
        ],
    )(xf, wt)

    return (weights_t.T.reshape(B, S, _TOP_K),
            idx_t.T.reshape(B, S, _TOP_K),
            probs_t.T.reshape(B, S, E))
